# Initial kernel scaffold; baseline (speedup 1.0000x reference)
import jax
import jax.numpy as jnp
from jax.experimental import pallas as pl


def kernel(embed, bias, W_enc, W_dec):
    raise NotImplementedError("write your pallas kernel here")


# speedup vs baseline: 4.2952x; 4.2952x over previous
import functools

import jax
import jax.numpy as jnp
from jax import lax
from jax.experimental import pallas as pl
from jax.experimental.pallas import tpu as pltpu
from jax.experimental.pallas import tpu_sc as plsc

B = 4096
E = 768
F = 65536
K = 32
GRP = 128
NGRP = F // GRP
FT = 512
NFT = F // FT

NC = 2
NS = 16
NW = NC * NS
RPW = B // NW

L = 16
GCAP = 192
CCAP = 512
RED = min(128, NGRP)
NRED = NGRP // RED
NEG = float("-inf")


def _tc_body(x_ref, b_ref, w_ref, proj_ref, m_ref):
    x = x_ref[...] - b_ref[...]
    w = w_ref[...]
    p = lax.dot_general(x, w, (((1,), (1,)), ((), ())),
                        preferred_element_type=jnp.float32)
    proj_ref[...] = p
    m_ref[0] = jnp.max(p.reshape(B, FT // GRP, GRP), axis=-1)


def _tc_project(embed, bias, W_enc, interpret=False):
    return pl.pallas_call(
        _tc_body,
        grid=(NFT,),
        in_specs=[
            pl.BlockSpec((B, E), lambda i: (0, 0)),
            pl.BlockSpec((E,), lambda i: (0,)),
            pl.BlockSpec((FT, E), lambda i: (i, 0)),
        ],
        out_specs=[
            pl.BlockSpec((B, FT), lambda i: (0, i)),
            pl.BlockSpec((1, B, FT // GRP), lambda i: (i, 0, 0)),
        ],
        out_shape=[
            jax.ShapeDtypeStruct((B, F), jnp.float32),
            jax.ShapeDtypeStruct((NFT, B, FT // GRP), jnp.float32),
        ],
        interpret=interpret,
    )(embed, bias, W_enc)


def _iota16():
    return lax.broadcasted_iota(jnp.int32, (L,), 0)


def _vmin(v):
    return lax.sort(v, dimension=0)[0]


def _vmax(v):
    return lax.sort(v, dimension=0)[L - 1]


def _sc_body(proj_hbm, m_hbm, wdec_hbm, bias_hbm,
             recon_hbm, tot_hbm,
             cnt_ref, m_ref, gid_ref, grp_ref, cv_ref, ci_ref,
             wsel_ref, fsel_ref, wrow_ref, acc_ref, bias_ref,
             siota_ref, shared_ref,
             gsem, wsem):
    cid = lax.axis_index("c")
    sid = lax.axis_index("s")
    wid = cid * NS + sid
    row0 = wid * RPW

    iota = _iota16()
    lane0 = iota == 0

    def splat_i(x):
        return jnp.broadcast_to(jnp.int32(x) * 1, (L,))

    def store1(ref, pos, val):
        plsc.store_scatter(ref, [jnp.broadcast_to(pos, (L,))],
                           jnp.broadcast_to(val, (L,)), mask=lane0)

    def load1(ref, pos):
        return plsc.load_gather(ref, [jnp.broadcast_to(pos, (L,))])[0]
    zeros_i = jnp.zeros((L,), jnp.int32)
    ones_i = jnp.ones((L,), jnp.int32)
    ninf = jnp.full((L,), NEG, jnp.float32)

    def zero_row(r, _):
        for j in range(GRP // L):
            cnt_ref[r, pl.ds(j * L, L)] = zeros_i
        return 0
    lax.fori_loop(0, NGRP, zero_row, 0)
    pltpu.sync_copy(bias_hbm, bias_ref)
    for j in range(NRED):
        for t in range(RED // L):
            siota_ref[j, pl.ds(t * L, L)] = iota + (j * RED + t * L)

    @pl.when(sid == 0)
    def _():
        pltpu.sync_copy(cnt_ref, shared_ref)
    plsc.subcore_barrier()

    def do_row(r, _):
        row = row0 + r
        pltpu.sync_copy(m_hbm.at[row], m_ref)

        m1 = ninf
        m2 = ninf
        for i in range(NGRP // L):
            v = m_ref[pl.ds(i * L, L)]
            lo = jnp.minimum(m1, v)
            m1 = jnp.maximum(m1, v)
            m2 = jnp.maximum(m2, lo)
        t_scalar = _vmin(m2)
        tv = jnp.broadcast_to(t_scalar, (L,))

        base = row * NGRP
        for j in range(GCAP // L):
            gid_ref[pl.ds(j * L, L)] = iota * 0 + base

        def gid_step(i, c):
            v = m_ref[pl.ds(i * L, L)]
            msk = v >= tv
            pos = plsc.cumsum(jnp.where(msk, ones_i, zeros_i)) + (c - 1)
            gid = iota + (base + i * L)
            mskb = msk & (pos < GCAP)
            plsc.store_scatter(gid_ref, [pos], gid, mask=mskb)
            return c + plsc.all_reduce_population_count(msk)[0]
        c = lax.fori_loop(0, NGRP // L, gid_step, jnp.int32(0))
        c = jnp.minimum(c, GCAP)
        nch = (c + L - 1) // L

        def fire(ch, _):
            pltpu.async_copy(
                proj_hbm.at[gid_ref.at[pl.ds(ch * L, L)]],
                grp_ref.at[pl.ds(ch * L, L)], gsem)
            return 0
        lax.fori_loop(0, nch, fire, 0)

        def drain(ch, _):
            pltpu.make_async_copy(
                proj_hbm.at[gid_ref.at[pl.ds(ch * L, L)]],
                grp_ref.at[pl.ds(ch * L, L)], gsem).wait()
            return 0
        lax.fori_loop(0, nch, drain, 0)

        for j in range(CCAP // L):
            cv_ref[pl.ds(j * L, L)] = ninf

        def chunk_step(ch, n):
            gidv = gid_ref[pl.ds(ch * L, L)]
            for j in range(L):
                g = ch * L + j
                cbase = (gidv[j] - base) * GRP
                valid = g < c

                def vreg_step(t, n, g=g, cbase=cbase, valid=valid):
                    v = grp_ref[g, pl.ds(t * L, L)]
                    msk = (v >= tv) & valid
                    pc = plsc.all_reduce_population_count(msk)[0]

                    def hit(n):
                        pos = (plsc.cumsum(jnp.where(msk, ones_i, zeros_i))
                               + (n - 1))
                        mskb = msk & (pos < CCAP)
                        col = iota + (cbase + t * L)
                        plsc.store_scatter(cv_ref, [pos], v, mask=mskb)
                        plsc.store_scatter(ci_ref, [pos], col, mask=mskb)
                        return n + pc
                    return lax.cond(pc > 0, hit, lambda n: n, n)
                n = lax.fori_loop(0, GRP // L, vreg_step, n)
            return n
        n = lax.fori_loop(0, nch, chunk_step, jnp.int32(0))
        n = jnp.minimum(n, CCAP)
        nv = (n + L - 1) // L

        for k in range(K):
            def vmax_step(j, mx):
                return jnp.maximum(mx, cv_ref[pl.ds(j * L, L)])
            mv = lax.fori_loop(0, nv, vmax_step, ninf)
            m_s = _vmax(mv)
            mvv = jnp.broadcast_to(m_s, (L,))

            def find_step(j, st):
                found, _ = st
                v = cv_ref[pl.ds(j * L, L)]
                msk = (v == mvv) & (found == 0)
                pc = plsc.all_reduce_population_count(msk)[0]

                def hit(st):
                    lane = plsc.all_reduce_ffs(msk)[0]
                    pos = j * L + lane
                    store1(cv_ref, pos, jnp.float32(NEG))
                    return (jnp.int32(1), pos)
                return lax.cond(pc > 0, hit, lambda st: st, st)
            _, pos = lax.fori_loop(0, nv, find_step,
                                   (jnp.int32(0), jnp.int32(0)))
            store1(wsel_ref, jnp.int32(k), m_s)
            store1(fsel_ref, jnp.int32(k), load1(ci_ref, pos))

        for t in range(K // L):
            f = fsel_ref[pl.ds(t * L, L)]
            hi = lax.shift_right_logical(f, 7)
            lo = jnp.bitwise_and(f, 127)
            plsc.addupdate_scatter(cnt_ref, [hi, lo], ones_i)

        cp = pltpu.async_copy(wdec_hbm.at[fsel_ref], wrow_ref, wsem)
        for j in range(E // L):
            acc_ref[pl.ds(j * L, L)] = bias_ref[pl.ds(j * L, L)]
        cp.wait()

        ws = [wsel_ref[pl.ds(t * L, L)] for t in range(K // L)]
        wsc = [ws[k // L][k % L] for k in range(K)]

        def dec_step(j, _):
            a = acc_ref[pl.ds(j * L, L)]
            for k in range(K):
                a = a + wrow_ref[k, pl.ds(j * L, L)] * wsc[k]
            acc_ref[pl.ds(j * L, L)] = a
            return 0
        lax.fori_loop(0, E // L, dec_step, 0)
        pltpu.sync_copy(acc_ref, recon_hbm.at[row])
        return 0

    lax.fori_loop(0, RPW, do_row, 0)

    plsc.subcore_barrier()
    for j in range(NRED):
        pltpu.sync_copy(cnt_ref.at[pl.ds(j * RED, RED)],
                        shared_ref.at[siota_ref.at[j]], add=True)
    plsc.subcore_barrier()

    @pl.when(sid == 0)
    def _():
        pltpu.sync_copy(shared_ref, tot_hbm.at[cid])


def _sc_stage(proj2d, m, W_dec, bias, interpret=False):
    mesh = plsc.VectorSubcoreMesh(core_axis_name="c", subcore_axis_name="s",
                                  num_cores=NC, num_subcores=NS)
    fn = pl.kernel(
        _sc_body,
        out_type=[
            jax.ShapeDtypeStruct((B, E), jnp.float32),
            jax.ShapeDtypeStruct((NC, NGRP, GRP), jnp.int32),
        ],
        mesh=mesh,
        scratch_types=[
            pltpu.VMEM((NGRP, GRP), jnp.int32),
            pltpu.VMEM((NGRP,), jnp.float32),
            pltpu.VMEM((GCAP,), jnp.int32),
            pltpu.VMEM((GCAP, GRP), jnp.float32),
            pltpu.VMEM((CCAP,), jnp.float32),
            pltpu.VMEM((CCAP,), jnp.int32),
            pltpu.VMEM((K,), jnp.float32),
            pltpu.VMEM((K,), jnp.int32),
            pltpu.VMEM((K, E), jnp.float32),
            pltpu.VMEM((E,), jnp.float32),
            pltpu.VMEM((E,), jnp.float32),
            pltpu.VMEM((NRED, RED), jnp.int32),
            pltpu.VMEM_SHARED((NGRP, GRP), jnp.int32),
            pltpu.SemaphoreType.DMA,
            pltpu.SemaphoreType.DMA,
        ],
        compiler_params=pltpu.CompilerParams(needs_layout_passes=False),
        interpret=interpret,
    )
    return fn(proj2d, m, W_dec, bias)


def kernel(embed, bias, W_enc, W_dec):
    proj, m3 = _tc_project(embed, bias, W_enc)
    proj2d = proj.reshape(B * NGRP, GRP)
    m = m3.swapaxes(0, 1).reshape(B, NGRP)
    recon, tot = _sc_stage(proj2d, m, W_dec, bias)
    total = tot[0].reshape(F) + tot[1].reshape(F)
    return recon, total
